# baseline (speedup 1.0000x reference)
import functools

import numpy as np
import jax
import jax.numpy as jnp
from jax import lax
from jax.experimental import pallas as pl
from jax.experimental.pallas import tpu as pltpu
from jax.experimental.pallas import tpu_sc as plsc

N_NODES = 100000
N_GRAPHS = 512
N_GEN = 16
C_DIM = 20
M_SYM = 128
OUT_FEATURES = 10

NC = 2
NS = 16
NW = NC * NS
N_PER_W = 3200
N_PAD = NW * N_PER_W
N_TAIL_OFF = (NW - 1) * N_PER_W
N_TAIL = N_NODES - N_TAIL_OFF
N_PW_PAD = N_PER_W - N_TAIL
VEC_GROUPS = 25
GROUP_ROWS = 8
HGROUPS = 66
HROWS = HGROUPS * GROUP_ROWS
ZROWS = HROWS // NS


def _contrastive_np() -> np.ndarray:
    pairs = [(i, j) for i in range(N_GEN) for j in range(i + 1, N_GEN)]
    mat = np.zeros((N_GEN, len(pairs)), dtype=np.float32)
    for k, (i, j) in enumerate(pairs):
        mat[i, k] = 1.0
        mat[j, k] = -1.0
    return mat


_CONTRASTIVE = _contrastive_np()
N_PAIRS = _CONTRASTIVE.shape[1]


@functools.cache
def _make_hist_sc():
    mesh = plsc.VectorSubcoreMesh(core_axis_name="c", subcore_axis_name="s")
    return pl.kernel(
        _hist_body,
        out_type=jax.ShapeDtypeStruct((NC, HROWS, M_SYM), jnp.float32),
        mesh=mesh,
        scratch_types=[
            pltpu.VMEM((N_PER_W,), jnp.int32),
            pltpu.VMEM((N_PER_W,), jnp.int32),
            pltpu.VMEM((N_PER_W,), jnp.int32),
            pltpu.VMEM((HGROUPS, GROUP_ROWS), jnp.int32),
            pltpu.VMEM((HROWS, M_SYM), jnp.float32),
            pltpu.VMEM((ZROWS, M_SYM), jnp.float32),
            pltpu.VMEM_SHARED((HROWS, M_SYM), jnp.float32),
            pltpu.SemaphoreType.DMA,
            pltpu.SemaphoreType.DMA,
        ],
        compiler_params=pltpu.CompilerParams(
            use_tc_tiling_on_sc=False, needs_layout_passes=False),
    )


def _hist_body(x_hbm, b_hbm, px_hbm, pb_hbm, rid_hbm, out_hbm,
               idx_v, bat_v, key_v, rid_v, hist, zbuf, sp_hist, sem, sem2):
    cid = lax.axis_index("c")
    sid = lax.axis_index("s")
    w = sid * NC + cid

    rcp = pltpu.async_copy(rid_hbm, rid_v, sem)

    @pl.when(w < NW - 1)
    def _():
        xcp = pltpu.async_copy(x_hbm.at[pl.ds(w * N_PER_W, N_PER_W)],
                               idx_v, sem)
        bcp = pltpu.async_copy(b_hbm.at[pl.ds(w * N_PER_W, N_PER_W)],
                               bat_v, sem)
        xcp.wait()
        bcp.wait()

    @pl.when(w == NW - 1)
    def _():
        xcp = pltpu.async_copy(x_hbm.at[pl.ds(N_TAIL_OFF, N_TAIL)],
                               idx_v.at[pl.ds(0, N_TAIL)], sem)
        bcp = pltpu.async_copy(b_hbm.at[pl.ds(N_TAIL_OFF, N_TAIL)],
                               bat_v.at[pl.ds(0, N_TAIL)], sem)
        pxcp = pltpu.async_copy(px_hbm, idx_v.at[pl.ds(N_TAIL, N_PW_PAD)], sem)
        pbcp = pltpu.async_copy(pb_hbm, bat_v.at[pl.ds(N_TAIL, N_PW_PAD)], sem)
        xcp.wait()
        bcp.wait()
        pxcp.wait()
        pbcp.wait()

    z16 = jnp.zeros((16,), jnp.float32)
    for r in range(ZROWS):
        for c in range(M_SYM // 16):
            zbuf[r, pl.ds(c * 16, 16)] = z16
    zcp = pltpu.async_copy(zbuf, sp_hist.at[pl.ds(sid * ZROWS, ZROWS)], sem2)

    rcp.wait()

    b_lo = jnp.min(bat_v[pl.ds(0, 16)])
    b_hi = jnp.max(bat_v[pl.ds(N_PER_W - 16, 16)])
    g_lo = b_lo // GROUP_ROWS
    g_hi = b_hi // GROUP_ROWS

    def zero_group(g, carry):
        for r in range(GROUP_ROWS):
            for c in range(M_SYM // 16):
                hist[g * GROUP_ROWS + r, pl.ds(c * 16, 16)] = z16
        return carry
    lax.fori_loop(g_lo, g_hi + 1, zero_group, 0)

    def sort_step(i, carry):
        base = i * 128
        for k in range(8):
            bv = bat_v[pl.ds(base + k * 16, 16)]
            xv = idx_v[pl.ds(base + k * 16, 16)]
            skey, _ = plsc.sort_key_val(bv * M_SYM + xv, xv)
            key_v[pl.ds(base + k * 16, 16)] = skey
        return carry
    lax.fori_loop(0, VEC_GROUPS, sort_step, 0)

    def hist_step(i, carry):
        base = i * 128
        for k in range(8):
            skey = key_v[pl.ds(base + k * 16, 16)]
            cnt, last = plsc.scan_count(skey)
            sb = lax.shift_right_logical(skey, 7)
            sx = lax.bitwise_and(skey, jnp.int32(M_SYM - 1))
            plsc.addupdate_scatter(
                hist, [sb, sx], cnt.astype(jnp.float32), mask=last)
        return carry
    lax.fori_loop(0, VEC_GROUPS, hist_step, 0)

    zcp.wait()
    plsc.subcore_barrier()

    def merge_group(g, carry):
        pltpu.async_copy(hist.at[pl.ds(g * GROUP_ROWS, GROUP_ROWS)],
                         sp_hist.at[rid_v.at[g]], sem, add=True)
        return carry
    lax.fori_loop(g_lo, g_hi + 1, merge_group, 0)

    def merge_drain(g, carry):
        pltpu.make_async_copy(hist.at[pl.ds(0, GROUP_ROWS)],
                              sp_hist.at[rid_v.at[0]], sem).wait()
        return carry
    lax.fori_loop(g_lo, g_hi + 1, merge_drain, 0)

    plsc.subcore_barrier()

    @pl.when(sid == 0)
    def _():
        pltpu.sync_copy(sp_hist, out_hbm.at[cid])


def _table_body(lp_ref, lb_ref, t_ref):
    a = lp_ref[:][:, :, None] + lb_ref[:]
    mx = jnp.max(a, axis=1)
    s = jnp.sum(jnp.exp(a - mx[:, None, :]), axis=1)
    t_ref[:] = (mx + jnp.log(s)).T


def _build_table(log_prior, log_B):
    return pl.pallas_call(
        _table_body,
        out_shape=jax.ShapeDtypeStruct((M_SYM, N_GEN), jnp.float32),
    )(log_prior, log_B)


def _head_body(hist_ref, t_ref, con_ref, wt_ref, b_ref, out_ref):
    h = hist_ref[0, :N_GRAPHS, :] + hist_ref[1, :N_GRAPHS, :]
    agg = jnp.dot(h, t_ref[:], preferred_element_type=jnp.float32,
                  precision=lax.Precision.HIGHEST)

    mean = jnp.mean(agg, axis=0, keepdims=True)
    var = jnp.mean((agg - mean) ** 2, axis=0, keepdims=True)
    bn = (agg - mean) / jnp.sqrt(var + 1e-5)
    c = jnp.tanh(jnp.dot(bn, con_ref[:], preferred_element_type=jnp.float32))
    out_ref[:] = (
        jnp.dot(c, wt_ref[:], preferred_element_type=jnp.float32) + b_ref[:]
    )


def _head(hist2, t, W, b):
    return pl.pallas_call(
        _head_body,
        out_shape=jax.ShapeDtypeStruct((N_GRAPHS, OUT_FEATURES), jnp.float32),
    )(hist2, t, _CONTRASTIVE, W.T, b.reshape(1, OUT_FEATURES))


_ROW_IDS = np.arange(HROWS, dtype=np.int32).reshape(HGROUPS, GROUP_ROWS)


def kernel(x, edge_index, batch, log_prior, log_B, W, b):
    del edge_index
    px = jnp.zeros((N_PW_PAD,), jnp.int32)
    pb = jnp.full((N_PW_PAD,), N_GRAPHS, jnp.int32)
    t = _build_table(log_prior, log_B)
    hist2 = _make_hist_sc()(x, batch, px, pb, _ROW_IDS)
    return _head(hist2, t, W, b)


# speedup vs baseline: 1.0104x; 1.0104x over previous
import functools

import numpy as np
import jax
import jax.numpy as jnp
from jax import lax
from jax.experimental import pallas as pl
from jax.experimental.pallas import tpu as pltpu
from jax.experimental.pallas import tpu_sc as plsc

N_NODES = 100000
N_GRAPHS = 512
N_GEN = 16
C_DIM = 20
M_SYM = 128
OUT_FEATURES = 10

NC = 2
NS = 16
NW = NC * NS
N_PER_W = 3200
N_PAD = NW * N_PER_W
N_TAIL_OFF = (NW - 1) * N_PER_W
N_TAIL = N_NODES - N_TAIL_OFF
N_PW_PAD = N_PER_W - N_TAIL
VEC_GROUPS = 25
GROUP_ROWS = 8
HGROUPS = 66
HROWS = HGROUPS * GROUP_ROWS
ZROWS = HROWS // NS


def _contrastive_np() -> np.ndarray:
    pairs = [(i, j) for i in range(N_GEN) for j in range(i + 1, N_GEN)]
    mat = np.zeros((N_GEN, len(pairs)), dtype=np.float32)
    for k, (i, j) in enumerate(pairs):
        mat[i, k] = 1.0
        mat[j, k] = -1.0
    return mat


_CONTRASTIVE = _contrastive_np()
N_PAIRS = _CONTRASTIVE.shape[1]


@functools.cache
def _make_hist_sc():
    mesh = plsc.VectorSubcoreMesh(core_axis_name="c", subcore_axis_name="s")
    return pl.kernel(
        _hist_body,
        out_type=jax.ShapeDtypeStruct((NC, HROWS, M_SYM), jnp.float32),
        mesh=mesh,
        scratch_types=[
            pltpu.VMEM((N_PER_W,), jnp.int32),
            pltpu.VMEM((N_PER_W,), jnp.int32),
            pltpu.VMEM((N_PER_W,), jnp.int32),
            pltpu.VMEM((HGROUPS, GROUP_ROWS), jnp.int32),
            pltpu.VMEM((HROWS, M_SYM), jnp.float32),
            pltpu.VMEM((ZROWS, M_SYM), jnp.float32),
            pltpu.VMEM_SHARED((HROWS, M_SYM), jnp.float32),
            pltpu.SemaphoreType.DMA,
            pltpu.SemaphoreType.DMA,
        ],
        compiler_params=pltpu.CompilerParams(
            use_tc_tiling_on_sc=False, needs_layout_passes=False),
    )


def _hist_body(x_hbm, b_hbm, px_hbm, pb_hbm, rid_hbm, out_hbm,
               idx_v, bat_v, key_v, rid_v, hist, zbuf, sp_hist, sem, sem2):
    cid = lax.axis_index("c")
    sid = lax.axis_index("s")
    w = sid * NC + cid

    rcp = pltpu.async_copy(rid_hbm, rid_v, sem)

    @pl.when(w < NW - 1)
    def _():
        xcp = pltpu.async_copy(x_hbm.at[pl.ds(w * N_PER_W, N_PER_W)],
                               idx_v, sem)
        bcp = pltpu.async_copy(b_hbm.at[pl.ds(w * N_PER_W, N_PER_W)],
                               bat_v, sem)
        xcp.wait()
        bcp.wait()

    @pl.when(w == NW - 1)
    def _():
        xcp = pltpu.async_copy(x_hbm.at[pl.ds(N_TAIL_OFF, N_TAIL)],
                               idx_v.at[pl.ds(0, N_TAIL)], sem)
        bcp = pltpu.async_copy(b_hbm.at[pl.ds(N_TAIL_OFF, N_TAIL)],
                               bat_v.at[pl.ds(0, N_TAIL)], sem)
        pxcp = pltpu.async_copy(px_hbm, idx_v.at[pl.ds(N_TAIL, N_PW_PAD)], sem)
        pbcp = pltpu.async_copy(pb_hbm, bat_v.at[pl.ds(N_TAIL, N_PW_PAD)], sem)
        xcp.wait()
        bcp.wait()
        pxcp.wait()
        pbcp.wait()

    z16 = jnp.zeros((16,), jnp.float32)
    for r in range(ZROWS):
        for c in range(M_SYM // 16):
            zbuf[r, pl.ds(c * 16, 16)] = z16
    zcp = pltpu.async_copy(zbuf, sp_hist.at[pl.ds(sid * ZROWS, ZROWS)], sem2)

    rcp.wait()

    b_lo = jnp.min(bat_v[pl.ds(0, 16)])
    b_hi = jnp.max(bat_v[pl.ds(N_PER_W - 16, 16)])
    g_lo = b_lo // GROUP_ROWS
    g_hi = b_hi // GROUP_ROWS

    def zero_group(g, carry):
        for r in range(GROUP_ROWS):
            for c in range(M_SYM // 16):
                hist[g * GROUP_ROWS + r, pl.ds(c * 16, 16)] = z16
        return carry
    lax.fori_loop(g_lo, g_hi + 1, zero_group, 0)

    def sort_step(i, carry):
        base = i * 128
        for k in range(8):
            bv = bat_v[pl.ds(base + k * 16, 16)]
            xv = idx_v[pl.ds(base + k * 16, 16)]
            skey, _ = plsc.sort_key_val(bv * M_SYM + xv, xv)
            key_v[pl.ds(base + k * 16, 16)] = skey
        return carry
    lax.fori_loop(0, VEC_GROUPS, sort_step, 0)

    def hist_step(i, carry):
        base = i * 128
        for k in range(8):
            skey = key_v[pl.ds(base + k * 16, 16)]
            cnt, last = plsc.scan_count(skey)
            sb = lax.shift_right_logical(skey, 7)
            sx = lax.bitwise_and(skey, jnp.int32(M_SYM - 1))
            plsc.addupdate_scatter(
                hist, [sb, sx], cnt.astype(jnp.float32), mask=last)
        return carry
    lax.fori_loop(0, VEC_GROUPS, hist_step, 0)

    zcp.wait()
    plsc.subcore_barrier()

    def merge_group(g, carry):
        pltpu.async_copy(hist.at[pl.ds(g * GROUP_ROWS, GROUP_ROWS)],
                         sp_hist.at[rid_v.at[g]], sem, add=True)
        return carry
    lax.fori_loop(g_lo, g_hi + 1, merge_group, 0)

    def merge_drain(g, carry):
        pltpu.make_async_copy(hist.at[pl.ds(0, GROUP_ROWS)],
                              sp_hist.at[rid_v.at[0]], sem).wait()
        return carry
    lax.fori_loop(g_lo, g_hi + 1, merge_drain, 0)

    plsc.subcore_barrier()

    @pl.when(sid == 0)
    def _():
        pltpu.sync_copy(sp_hist, out_hbm.at[cid])


def _table_body(lp_ref, lb_ref, t_ref):
    a = lp_ref[:][:, :, None] + lb_ref[:]
    mx = jnp.max(a, axis=1)
    s = jnp.sum(jnp.exp(a - mx[:, None, :]), axis=1)
    t_ref[:] = (mx + jnp.log(s)).T


def _build_table(log_prior, log_B):
    return pl.pallas_call(
        _table_body,
        out_shape=jax.ShapeDtypeStruct((M_SYM, N_GEN), jnp.float32),
    )(log_prior, log_B)


def _head_body(hist_ref, t_ref, con_ref, w_ref, b_ref, out_ref):
    h = hist_ref[0, :N_GRAPHS, :] + hist_ref[1, :N_GRAPHS, :]
    agg = jnp.dot(h, t_ref[:], preferred_element_type=jnp.float32,
                  precision=lax.Precision.HIGHEST)

    mean = jnp.mean(agg, axis=0, keepdims=True)
    var = jnp.mean((agg - mean) ** 2, axis=0, keepdims=True)
    bn = (agg - mean) / jnp.sqrt(var + 1e-5)
    c = jnp.tanh(jnp.dot(bn, con_ref[:], preferred_element_type=jnp.float32))
    out_ref[:] = lax.dot_general(
        c, w_ref[:], (((1,), (1,)), ((), ())),
        preferred_element_type=jnp.float32) + b_ref[:]


def _head(hist2, t, W, b):
    return pl.pallas_call(
        _head_body,
        out_shape=jax.ShapeDtypeStruct((N_GRAPHS, OUT_FEATURES), jnp.float32),
    )(hist2, t, _CONTRASTIVE, W, b.reshape(1, OUT_FEATURES))


_ROW_IDS = np.arange(HROWS, dtype=np.int32).reshape(HGROUPS, GROUP_ROWS)
_PAD_X = np.zeros((N_PW_PAD,), np.int32)
_PAD_B = np.full((N_PW_PAD,), N_GRAPHS, np.int32)


def kernel(x, edge_index, batch, log_prior, log_B, W, b):
    del edge_index
    px = _PAD_X
    pb = _PAD_B
    t = _build_table(log_prior, log_B)
    hist2 = _make_hist_sc()(x, batch, px, pb, _ROW_IDS)
    return _head(hist2, t, W, b)
